# baseline (speedup 1.0000x reference)
import functools

import jax
import jax.numpy as jnp
from jax import lax
from jax.experimental import pallas as pl
from jax.experimental.pallas import tpu as pltpu
from jax.experimental.pallas import tpu_sc as plsc

N = 2048
_SB = 256
_FB = 256
_EPS = 1e-8
_BIG = 2**30

_NIDX = 3 * N
_NW = 32
_CHUNK = 96
_CPW = _NIDX // (_NW * _CHUNK)


@functools.partial(
    pl.kernel,
    mesh=plsc.VectorSubcoreMesh(core_axis_name="c", subcore_axis_name="s"),
    out_type=jax.ShapeDtypeStruct((_NIDX, 128), jnp.float32),
    scratch_types=[
        pltpu.VMEM((_CPW, _CHUNK), jnp.int32),
        pltpu.VMEM((_CPW * _CHUNK, 128), jnp.float32),
        pltpu.SemaphoreType.DMA,
    ],
)
def _sc_gather(table_hbm, idx_hbm, out_hbm, idx_v, rows_v, sem):
    wid = lax.axis_index("s") * 2 + lax.axis_index("c")
    pltpu.sync_copy(idx_hbm.at[pl.ds(wid * _CPW, _CPW)], idx_v)
    copies = [
        pltpu.async_copy(table_hbm.at[idx_v.at[j]],
                         rows_v.at[pl.ds(j * _CHUNK, _CHUNK)], sem)
        for j in range(_CPW)
    ]
    for c in copies:
        c.wait()
    pltpu.sync_copy(rows_v, out_hbm.at[pl.ds(wid * _CPW * _CHUNK,
                                             _CPW * _CHUNK)])


def _dot(a, b):
    return jax.lax.dot_general(
        a, b, (((1,), (0,)), ((), ())),
        precision=jax.lax.Precision.HIGHEST,
        preferred_element_type=jnp.float32)


def _body(obj_t_ref, sr_ref, fv_ref, normals_ref, pen_ref, cmap_ref):
    ox = obj_t_ref[0:1, :]
    oy = obj_t_ref[1:2, :]
    oz = obj_t_ref[2:3, :]
    obj3 = obj_t_ref[0:3, :]
    o2 = ox * ox + oy * oy + oz * oz

    def nn_step(b, carry):
        m, idx = carry
        base = b * _SB
        s_blk = sr_ref[pl.ds(base, _SB), 0:3]
        g2 = _dot(s_blk * -2.0, obj3)
        s2 = jnp.sum(s_blk * s_blk, axis=1, keepdims=True)
        d2 = (s2 + g2) + o2
        bmin = jnp.min(d2, axis=0, keepdims=True)
        iota = lax.broadcasted_iota(jnp.int32, (_SB, N), 0) + base
        barg = jnp.min(jnp.where(d2 == bmin, iota, _BIG), axis=0,
                       keepdims=True)
        take = bmin < m
        return jnp.where(take, bmin, m), jnp.where(take, barg, idx)

    m0 = jnp.full((1, N), jnp.inf, jnp.float32)
    i0 = jnp.zeros((1, N), jnp.int32)
    nn_d, nn_idx = lax.fori_loop(0, N // _SB, nn_step, (m0, i0))

    cmap_ref[0:1, :] = 1.0 - 2.0 * (jax.nn.sigmoid(100.0 * nn_d) - 0.5)

    def trg_step(b, carry):
        tx, ty, tz = carry
        base = b * _SB
        iota = lax.broadcasted_iota(jnp.int32, (_SB, N), 0) + base
        sel = iota == nn_idx
        sx = sr_ref[pl.ds(base, _SB), 0:1]
        sy = sr_ref[pl.ds(base, _SB), 1:2]
        sz = sr_ref[pl.ds(base, _SB), 2:3]
        tx = tx + jnp.sum(jnp.where(sel, sx, 0.0), axis=0, keepdims=True)
        ty = ty + jnp.sum(jnp.where(sel, sy, 0.0), axis=0, keepdims=True)
        tz = tz + jnp.sum(jnp.where(sel, sz, 0.0), axis=0, keepdims=True)
        return tx, ty, tz

    z0 = jnp.zeros((1, N), jnp.float32)
    tx, ty, tz = lax.fori_loop(0, N // _SB, trg_step, (z0, z0, z0))

    dxr = tx - ox
    dyr = ty - oy
    dzr = tz - oz
    rays = jnp.concatenate(
        [jnp.concatenate([dxr, dyr, dzr], axis=0), obj3], axis=1)

    def face_step(b, counts):
        base = b * _FB
        v0 = fv_ref[pl.ds(base, _FB), 0:3]
        v1 = fv_ref[pl.ds(N + base, _FB), 0:3]
        v2 = fv_ref[pl.ds(2 * N + base, _FB), 0:3]
        nrm = normals_ref[pl.ds(base, _FB), 0:3]
        e0 = v1 - v0
        e1 = v2 - v0

        g_n = _dot(nrm, rays)
        g_e0 = _dot(e0, rays)
        g_e1 = _dot(e1, rays)
        denom = g_n[:, :N]
        no_ = g_n[:, N:]
        e0d = g_e0[:, :N]
        e0o = g_e0[:, N:]
        e1d = g_e1[:, :N]
        e1o = g_e1[:, N:]

        nv0 = jnp.sum(nrm * v0, axis=1, keepdims=True)
        e0v0 = jnp.sum(e0 * v0, axis=1, keepdims=True)
        e1v0 = jnp.sum(e1 * v0, axis=1, keepdims=True)
        dot00 = jnp.sum(e0 * e0, axis=1, keepdims=True)
        dot01 = jnp.sum(e0 * e1, axis=1, keepdims=True)
        dot11 = jnp.sum(e1 * e1, axis=1, keepdims=True)

        valid = jnp.abs(denom) > _EPS
        safe_denom = jnp.where(valid, denom, 1.0)
        t = (nv0 - no_) / safe_denom
        dot0w = (e0o - e0v0) + t * e0d
        dot1w = (e1o - e1v0) + t * e1d
        den = dot00 * dot11 - dot01 * dot01
        rcp = 1.0 / jnp.where(jnp.abs(den) > _EPS, den, 1.0)
        u = (dot11 * dot0w - dot01 * dot1w) * rcp
        v = (dot00 * dot1w - dot01 * dot0w) * rcp
        inside = (u >= -_EPS) & (v >= -_EPS) & (u + v <= 1.0 + _EPS)
        hit = valid & (t > _EPS) & inside
        return counts + jnp.sum(hit.astype(jnp.int32), axis=0, keepdims=True)

    counts = lax.fori_loop(0, N // _FB, face_step,
                           jnp.zeros((1, N), jnp.int32))

    interior = (counts % 2) != 0
    pen2 = jnp.sum(jnp.where(interior, nn_d, 0.0), axis=1, keepdims=True)
    pen_ref[0:1, 0:1] = jnp.sqrt(pen2)


def kernel(obj_points, sr_points, hand_verts, hand_faces, face_normals):
    obj_t = obj_points.T
    faces_idx = hand_faces.astype(jnp.int32).T.reshape(_NW * _CPW, _CHUNK)
    verts_pad = jnp.pad(hand_verts, ((0, 0), (0, 125)))
    fv = _sc_gather(verts_pad, faces_idx)
    pen, cmap = pl.pallas_call(
        _body,
        out_shape=[
            jax.ShapeDtypeStruct((1, 1), jnp.float32),
            jax.ShapeDtypeStruct((1, N), jnp.float32),
        ],
    )(obj_t, sr_points, fv, face_normals)
    return pen[0, 0], cmap[0]


# speedup vs baseline: 1.6775x; 1.6775x over previous
import jax
import jax.numpy as jnp
from jax import lax
from jax.experimental import pallas as pl

N = 2048
_SB = 256
_FB = 256
_EPS = 1e-8
_BIG = 2**30


def _body(obj_t_ref, sr_ref, verts_t_ref, faces_ref, normals_ref,
          pen_ref, cmap_ref):
    ox = obj_t_ref[0:1, :]
    oy = obj_t_ref[1:2, :]
    oz = obj_t_ref[2:3, :]
    iota_s = lax.broadcasted_iota(jnp.int32, (_SB, N), 0)

    def nn_step(b, carry):
        m, idx = carry
        base = b * _SB
        sx = sr_ref[pl.ds(base, _SB), 0:1]
        sy = sr_ref[pl.ds(base, _SB), 1:2]
        sz = sr_ref[pl.ds(base, _SB), 2:3]
        ddx = ox - sx
        ddy = oy - sy
        ddz = oz - sz
        d2 = ddx * ddx + ddy * ddy + ddz * ddz
        bmin = jnp.min(d2, axis=0, keepdims=True)
        barg = jnp.min(jnp.where(d2 == bmin, iota_s, _BIG), axis=0,
                       keepdims=True) + base
        take = bmin < m
        return jnp.where(take, bmin, m), jnp.where(take, barg, idx)

    m0 = jnp.full((1, N), jnp.inf, jnp.float32)
    i0 = jnp.zeros((1, N), jnp.int32)
    nn_d, nn_idx = lax.fori_loop(0, N // _SB, nn_step, (m0, i0))

    cmap_ref[0:1, :] = 1.0 - 2.0 * (jax.nn.sigmoid(100.0 * nn_d) - 0.5)

    def trg_step(b, carry):
        tx, ty, tz = carry
        base = b * _SB
        sel = iota_s == (nn_idx - base)
        sx = sr_ref[pl.ds(base, _SB), 0:1]
        sy = sr_ref[pl.ds(base, _SB), 1:2]
        sz = sr_ref[pl.ds(base, _SB), 2:3]
        tx = tx + jnp.sum(jnp.where(sel, sx, 0.0), axis=0, keepdims=True)
        ty = ty + jnp.sum(jnp.where(sel, sy, 0.0), axis=0, keepdims=True)
        tz = tz + jnp.sum(jnp.where(sel, sz, 0.0), axis=0, keepdims=True)
        return tx, ty, tz

    z0 = jnp.zeros((1, N), jnp.float32)
    tx, ty, tz = lax.fori_loop(0, N // _SB, trg_step, (z0, z0, z0))

    dxr = tx - ox
    dyr = ty - oy
    dzr = tz - oz

    vx = verts_t_ref[0:1, :]
    vy = verts_t_ref[1:2, :]
    vz = verts_t_ref[2:3, :]
    lane_f = lax.broadcasted_iota(jnp.int32, (_FB, N), 1)

    def face_step(b, counts):
        base = b * _FB

        def gather(col):
            fi = faces_ref[pl.ds(base, _FB), col:col + 1]
            sel = lane_f == fi
            gx = jnp.sum(jnp.where(sel, vx, 0.0), axis=1, keepdims=True)
            gy = jnp.sum(jnp.where(sel, vy, 0.0), axis=1, keepdims=True)
            gz = jnp.sum(jnp.where(sel, vz, 0.0), axis=1, keepdims=True)
            return gx, gy, gz

        v0x, v0y, v0z = gather(0)
        v1x, v1y, v1z = gather(1)
        v2x, v2y, v2z = gather(2)
        nx = normals_ref[pl.ds(base, _FB), 0:1]
        ny = normals_ref[pl.ds(base, _FB), 1:2]
        nz = normals_ref[pl.ds(base, _FB), 2:3]

        denom = nx * dxr + ny * dyr + nz * dzr
        valid = jnp.abs(denom) > _EPS
        safe_denom = jnp.where(valid, denom, 1.0)
        t = (nx * (v0x - ox) + ny * (v0y - oy) + nz * (v0z - oz)) / safe_denom
        px = ox + t * dxr
        py = oy + t * dyr
        pz = oz + t * dzr
        wx = px - v0x
        wy = py - v0y
        wz = pz - v0z
        e0x = v1x - v0x
        e0y = v1y - v0y
        e0z = v1z - v0z
        e1x = v2x - v0x
        e1y = v2y - v0y
        e1z = v2z - v0z
        dot00 = e0x * e0x + e0y * e0y + e0z * e0z
        dot01 = e0x * e1x + e0y * e1y + e0z * e1z
        dot11 = e1x * e1x + e1y * e1y + e1z * e1z
        dot0w = e0x * wx + e0y * wy + e0z * wz
        dot1w = e1x * wx + e1y * wy + e1z * wz
        den = dot00 * dot11 - dot01 * dot01
        rcp = 1.0 / jnp.where(jnp.abs(den) > _EPS, den, 1.0)
        u = (dot11 * dot0w - dot01 * dot1w) * rcp
        v = (dot00 * dot1w - dot01 * dot0w) * rcp
        inside = (u >= -_EPS) & (v >= -_EPS) & (u + v <= 1.0 + _EPS)
        hit = valid & (t > _EPS) & inside
        return counts + jnp.sum(hit.astype(jnp.int32), axis=0, keepdims=True)

    counts = lax.fori_loop(0, N // _FB, face_step,
                           jnp.zeros((1, N), jnp.int32))

    interior = (counts % 2) != 0
    pen2 = jnp.sum(jnp.where(interior, nn_d, 0.0), axis=1, keepdims=True)
    pen_ref[0:1, 0:1] = jnp.sqrt(pen2)


def kernel(obj_points, sr_points, hand_verts, hand_faces, face_normals):
    obj_t = obj_points.T
    verts_t = hand_verts.T
    faces = hand_faces.astype(jnp.int32)
    pen, cmap = pl.pallas_call(
        _body,
        out_shape=[
            jax.ShapeDtypeStruct((1, 1), jnp.float32),
            jax.ShapeDtypeStruct((1, N), jnp.float32),
        ],
    )(obj_t, sr_points, verts_t, faces, face_normals)
    return pen[0, 0], cmap[0]
